# baseline (speedup 1.0000x reference)
import jax
import jax.numpy as jnp
from jax.experimental import pallas as pl

_K = 8


def _knn_kernel(x_ref, ei_ref, ew_ref, *, b, n):
    c_iota_i = jax.lax.broadcasted_iota(jnp.int32, (n, n), 1)
    r_iota_i = jax.lax.broadcasted_iota(jnp.int32, (n, n), 0)
    neg_inf = jnp.float32(-jnp.inf)
    c_iota = c_iota_i.astype(jnp.float32)
    diag = r_iota_i == c_iota_i
    nf = jnp.float32(n)
    src = jax.lax.broadcasted_iota(jnp.int32, (n, _K), 0)

    for bi in range(b):
        x = x_ref[bi]
        norm = jnp.sqrt(jnp.sum(x * x, axis=1, keepdims=True))
        fn = x / jnp.maximum(norm, 1e-12)
        sim = jax.lax.dot_general(
            fn, fn, (((1,), (1,)), ((), ())),
            preferred_element_type=jnp.float32,
        )
        sim = jnp.where(diag, neg_inf, sim)

        for j in range(_K):
            m = jnp.max(sim, axis=1, keepdims=True)
            ismax = sim == m
            idxf = jnp.min(jnp.where(ismax, c_iota, nf), axis=1, keepdims=True)
            ew_ref[bi, :, j] = m[:, 0]
            ei_ref[bi, 1, :, j] = idxf[:, 0].astype(jnp.int32)
            if j + 1 < _K:
                sim = jnp.where(c_iota == idxf, neg_inf, sim)

        ei_ref[bi, 0, :, :] = src


def kernel(node_features):
    b, n, c = node_features.shape

    ei, ew = pl.pallas_call(
        lambda x_ref, ei_ref, ew_ref: _knn_kernel(
            x_ref, ei_ref, ew_ref, b=b, n=n),
        out_shape=[
            jax.ShapeDtypeStruct((b, 2, n, _K), jnp.int32),
            jax.ShapeDtypeStruct((b, n, _K), jnp.float32),
        ],
    )(node_features)

    edge_index = ei.reshape(b, 2, n * _K)
    edge_weight = ew.reshape(b, n * _K)
    return edge_index, edge_weight


# speedup vs baseline: 1.0776x; 1.0776x over previous
import jax
import jax.numpy as jnp
from jax.experimental import pallas as pl

_K = 8


def _knn_kernel(x_ref, ei_ref, ew_ref, *, n):
    x = x_ref[0]
    norm = jnp.sqrt(jnp.sum(x * x, axis=1, keepdims=True))
    fn = x / jnp.maximum(norm, 1e-12)
    sim = jax.lax.dot_general(
        fn, fn, (((1,), (1,)), ((), ())),
        preferred_element_type=jnp.float32,
    )

    c_iota_i = jax.lax.broadcasted_iota(jnp.int32, (n, n), 1)
    r_iota_i = jax.lax.broadcasted_iota(jnp.int32, (n, n), 0)
    neg_inf = jnp.float32(-jnp.inf)
    sim = jnp.where(r_iota_i == c_iota_i, neg_inf, sim)
    c_iota = c_iota_i.astype(jnp.float32)

    nf = jnp.float32(n)
    for j in range(_K):
        m = jnp.max(sim, axis=1, keepdims=True)
        ismax = sim == m
        idxf = jnp.min(jnp.where(ismax, c_iota, nf), axis=1, keepdims=True)
        ew_ref[0, :, j] = m[:, 0]
        ei_ref[0, 1, :, j] = idxf[:, 0].astype(jnp.int32)
        if j + 1 < _K:
            sim = jnp.where(c_iota == idxf, neg_inf, sim)

    src = jax.lax.broadcasted_iota(jnp.int32, (n, _K), 0)
    ei_ref[0, 0, :, :] = src


def kernel(node_features):
    b, n, c = node_features.shape

    ei, ew = pl.pallas_call(
        lambda x_ref, ei_ref, ew_ref: _knn_kernel(x_ref, ei_ref, ew_ref, n=n),
        grid=(b,),
        in_specs=[pl.BlockSpec((1, n, c), lambda i: (i, 0, 0))],
        out_specs=[
            pl.BlockSpec((1, 2, n, _K), lambda i: (i, 0, 0, 0)),
            pl.BlockSpec((1, n, _K), lambda i: (i, 0, 0)),
        ],
        out_shape=[
            jax.ShapeDtypeStruct((b, 2, n, _K), jnp.int32),
            jax.ShapeDtypeStruct((b, n, _K), jnp.float32),
        ],
    )(node_features)

    edge_index = ei.reshape(b, 2, n * _K)
    edge_weight = ew.reshape(b, n * _K)
    return edge_index, edge_weight
